# Initial kernel scaffold; baseline (speedup 1.0000x reference)
import jax
import jax.numpy as jnp
from jax.experimental import pallas as pl


def kernel(node_feature, edge_index, W, a_src, a_dst):
    raise NotImplementedError("write your pallas kernel here")


# speedup vs baseline: 19.8705x; 19.8705x over previous
import jax
import jax.numpy as jnp
from jax import lax
from jax.experimental import pallas as pl
from jax.experimental.pallas import tpu as pltpu
from jax.experimental.pallas import tpu_sc as plsc

N = 10000
NPAD = 10240
E = 320000
D = 128
NC = 2
NS = 16
NW = NC * NS
EPT = E // NW
CHUNK = 64
NCH = EPT // CHUNK
TAILB = EPT - NCH * CHUNK
ROWS_PER_SUB = NPAD // NS


def _prep_body(x_ref, w_ref, asr_ref, adr_ref, h_ref, es_ref, ed_ref):
    h = jnp.dot(x_ref[...], w_ref[...], preferred_element_type=jnp.float32)
    h_ref[...] = h
    es_ref[...] = jnp.sum(h * asr_ref[...], axis=1)
    ed_ref[...] = jnp.sum(h * adr_ref[...], axis=1)


def _tc_prep(x_pad, w, a_src, a_dst):
    B = 1024
    grid = NPAD // B
    return pl.pallas_call(
        _prep_body,
        grid=(grid,),
        in_specs=[
            pl.BlockSpec((B, D), lambda i: (i, 0)),
            pl.BlockSpec((D, D), lambda i: (0, 0)),
            pl.BlockSpec((1, D), lambda i: (0, 0)),
            pl.BlockSpec((1, D), lambda i: (0, 0)),
        ],
        out_specs=[
            pl.BlockSpec((B, D), lambda i: (i, 0)),
            pl.BlockSpec((B,), lambda i: (i,)),
            pl.BlockSpec((B,), lambda i: (i,)),
        ],
        out_shape=[
            jax.ShapeDtypeStruct((NPAD, D), jnp.float32),
            jax.ShapeDtypeStruct((NPAD,), jnp.float32),
            jax.ShapeDtypeStruct((NPAD,), jnp.float32),
        ],
    )(x_pad, w, a_src.reshape(1, D), a_dst.reshape(1, D))


def _edge_group(es_v, ed_v, s_v, ex_v, src_v, dst_v, g, cvec):
    sidx = src_v[0, pl.ds(g * 16, 16)]
    didx = dst_v[0, pl.ds(g * 16, 16)]
    a = plsc.load_gather(es_v, [sidx])
    b = plsc.load_gather(ed_v, [didx])
    x = a + b
    e = jnp.where(x >= 0.0, x, 0.2 * x)
    ex = jnp.exp(e - cvec)
    plsc.addupdate_scatter(s_v, [didx], ex)
    ex_v[pl.ds(16 + g * 16, 16)] = ex


def _scale_rows(rows_v, ex_v, n):
    for i in range(n):
        exb = plsc.load_gather(ex_v, [jnp.full((16,), 16 + i, jnp.int32)])
        for c in range(D // 16):
            sl = pl.ds(c * 16, 16)
            rows_v[i, sl] = rows_v[i, sl] * exb


def _sc_body(h_hbm, es_hbm, ed_hbm, src_hbm, dst_hbm, cvec_hbm,
             outp_hbm, sp_hbm,
             es_v, ed_v, s_v, cvec_v, src_v, dst_v, ex_v, rows_v,
             src_t, dst_t, ex_t, rows_t, zrow_v, out_sh, sem):
    cid = lax.axis_index("c")
    sid = lax.axis_index("s")
    wid = sid * NC + cid
    ebase = wid * EPT

    pltpu.sync_copy(es_hbm, es_v)
    pltpu.sync_copy(ed_hbm, ed_v)
    pltpu.sync_copy(cvec_hbm, cvec_v)
    cvec = cvec_v[...]

    def _z16(i, _):
        s_v[pl.ds(i * 16, 16)] = jnp.zeros((16,), jnp.float32)
        return 0
    lax.fori_loop(0, NPAD // 16, _z16, 0)

    for r in range(16):
        for c in range(D // 16):
            zrow_v[r, pl.ds(c * 16, 16)] = jnp.zeros((16,), jnp.float32)

    row0 = sid * ROWS_PER_SUB
    def _zslab(k, _):
        pltpu.sync_copy(zrow_v, out_sh.at[pl.ds(row0 + k * 16, 16), :])
        return 0
    lax.fori_loop(0, ROWS_PER_SUB // 16, _zslab, 0)
    plsc.subcore_barrier()

    def _chunk(i, _):
        eb = ebase + i * CHUNK
        pltpu.sync_copy(src_hbm.at[pl.ds(eb, CHUNK)], src_v.at[0])
        pltpu.sync_copy(dst_hbm.at[pl.ds(eb, CHUNK)], dst_v.at[0])
        for g in range(CHUNK // 16):
            _edge_group(es_v, ed_v, s_v, ex_v, src_v, dst_v, g, cvec)
        pltpu.async_copy(h_hbm.at[src_v.at[0]], rows_v, sem).wait()
        _scale_rows(rows_v, ex_v, CHUNK)
        pltpu.sync_copy(rows_v, out_sh.at[dst_v.at[0]], add=True)
        return 0
    lax.fori_loop(0, NCH, _chunk, 0)

    eb = ebase + NCH * CHUNK
    pltpu.sync_copy(src_hbm.at[pl.ds(eb, TAILB)], src_t.at[0])
    pltpu.sync_copy(dst_hbm.at[pl.ds(eb, TAILB)], dst_t.at[0])
    _edge_group(es_v, ed_v, s_v, ex_t, src_t, dst_t, 0, cvec)
    pltpu.async_copy(h_hbm.at[src_t.at[0]], rows_t, sem).wait()
    _scale_rows(rows_t, ex_t, TAILB)
    pltpu.sync_copy(rows_t, out_sh.at[dst_t.at[0]], add=True)

    pltpu.sync_copy(s_v, sp_hbm.at[wid])
    plsc.subcore_barrier()
    def _out(k, _):
        r = row0 + k * 16
        pltpu.sync_copy(out_sh.at[pl.ds(r, 16), :], outp_hbm.at[cid, pl.ds(r, 16), :])
        return 0
    lax.fori_loop(0, ROWS_PER_SUB // 16, _out, 0)


def _sc_edges(h, es, ed, src, dst, cvec):
    mesh = plsc.VectorSubcoreMesh(
        core_axis_name="c", subcore_axis_name="s", num_cores=NC, num_subcores=NS)
    f = pl.kernel(
        _sc_body,
        out_type=[
            jax.ShapeDtypeStruct((NC, NPAD, D), jnp.float32),
            jax.ShapeDtypeStruct((NW, NPAD), jnp.float32),
        ],
        mesh=mesh,
        compiler_params=pltpu.CompilerParams(needs_layout_passes=False),
        scratch_types=[
            pltpu.VMEM((NPAD,), jnp.float32),
            pltpu.VMEM((NPAD,), jnp.float32),
            pltpu.VMEM((NPAD,), jnp.float32),
            pltpu.VMEM((16,), jnp.float32),
            pltpu.VMEM((1, CHUNK), jnp.int32),
            pltpu.VMEM((1, CHUNK), jnp.int32),
            pltpu.VMEM((16 + CHUNK,), jnp.float32),
            pltpu.VMEM((CHUNK, D), jnp.float32),
            pltpu.VMEM((1, TAILB), jnp.int32),
            pltpu.VMEM((1, TAILB), jnp.int32),
            pltpu.VMEM((16 + TAILB,), jnp.float32),
            pltpu.VMEM((TAILB, D), jnp.float32),
            pltpu.VMEM((16, D), jnp.float32),
            pltpu.VMEM_SHARED((NPAD, D), jnp.float32),
            pltpu.SemaphoreType.DMA,
        ],
    )
    return f(h, es, ed, src, dst, cvec)


def _comb_body(outp_ref, sp_ref, o_ref):
    s = jnp.sum(sp_ref[...], axis=0) + 1e-16
    p = outp_ref[0] + outp_ref[1]
    o_ref[...] = p / s[:, None]


def _tc_combine(outp, sp):
    B = 1024
    grid = NPAD // B
    return pl.pallas_call(
        _comb_body,
        grid=(grid,),
        in_specs=[
            pl.BlockSpec((NC, B, D), lambda i: (0, i, 0)),
            pl.BlockSpec((NW, B), lambda i: (0, i)),
        ],
        out_specs=pl.BlockSpec((B, D), lambda i: (i, 0)),
        out_shape=jax.ShapeDtypeStruct((NPAD, D), jnp.float32),
    )(outp, sp)


@jax.jit
def kernel(node_feature, edge_index, W, a_src, a_dst):
    x_pad = jnp.zeros((NPAD, D), jnp.float32).at[:N].set(node_feature)
    h, es, ed = _tc_prep(x_pad, W, a_src, a_dst)
    c = jnp.maximum(jnp.max(es) + jnp.max(ed), 0.0)
    cvec = jnp.full((16,), c, jnp.float32)
    outp, sp = _sc_edges(h, es, ed, edge_index[0], edge_index[1], cvec)
    out = _tc_combine(outp, sp)
    return out[:N]
